# Initial kernel scaffold; baseline (speedup 1.0000x reference)
import jax
import jax.numpy as jnp
from jax.experimental import pallas as pl


def kernel(distance):
    raise NotImplementedError("write your pallas kernel here")


# speedup vs baseline: 21.4154x; 21.4154x over previous
import jax
import jax.numpy as jnp
from jax.experimental import pallas as pl


def _distance_norm_block(x_ref, o_ref):
    x = x_ref[0]
    L, D = x.shape
    Df = float(D)

    px = jnp.sum(x, axis=0, keepdims=True)
    px = px / jnp.sum(px)

    rng = jax.lax.broadcasted_iota(jnp.int32, (1, D), 1).astype(jnp.float32) - (D // 2) + 1.0
    mean = jnp.sum(px * rng)
    std = jnp.sqrt(jnp.sum(px * jnp.square(rng - mean)))

    new_indices = rng * (std / (Df * 0.1)) + mean + (Df / 2.0 - 1.0)
    idx = jnp.clip(new_indices + 1.0, 0.0, Df + 1.0)
    floor_i = idx.astype(jnp.int32)
    w = idx - floor_i.astype(jnp.float32)

    c1 = jax.lax.broadcasted_iota(jnp.int32, (D, D), 0) + 1
    fl = jnp.broadcast_to(floor_i, (D, D))
    wb = jnp.broadcast_to(w, (D, D))
    m = jnp.where(fl == c1, 1.0 - wb, 0.0) + jnp.where(fl + 1 == c1, wb, 0.0)

    o_ref[0] = jnp.dot(x, m, preferred_element_type=jnp.float32)


def kernel(distance):
    orig_shape = distance.shape
    L, D = orig_shape[-2], orig_shape[-1]
    x = distance.reshape(-1, L, D)
    B = x.shape[0]
    out = pl.pallas_call(
        _distance_norm_block,
        grid=(B,),
        in_specs=[pl.BlockSpec((1, L, D), lambda b: (b, 0, 0))],
        out_specs=pl.BlockSpec((1, L, D), lambda b: (b, 0, 0)),
        out_shape=jax.ShapeDtypeStruct((B, L, D), jnp.float32),
    )(x)
    return out.reshape(orig_shape)
